# baseline (speedup 1.0000x reference)
import jax
import jax.numpy as jnp
from jax import lax
from jax.experimental import pallas as pl
from jax.experimental.pallas import tpu as pltpu
from jax.experimental.pallas import tpu_sc as plsc

_VOCAB = 1000
_W = 26
_ROW = _W * _VOCAB
_NC = 2
_NS = 16
_NW = _NC * _NS
_RPC = 2
_BATCH = 4096
_ROWS_PER_TILE = _BATCH // _NW
_CHUNKS = _ROWS_PER_TILE // _RPC
_ENT = _RPC * _W
_ENT_PAD = 64
_CHUNK_F32 = _RPC * _ROW
_SLOT = _CHUNK_F32 + 16
_VBUF = 2 * _SLOT


def _sc_body(pos_hbm, out_hbm, pos_vmem, vbuf, sem0, sem1):
    cid = lax.axis_index("c")
    sid = lax.axis_index("s")
    wid = sid * _NC + cid
    elem_base = wid * _ROWS_PER_TILE * _ROW
    sems = (sem0, sem1)

    pltpu.sync_copy(pos_hbm.at[wid], pos_vmem)

    zeros16 = jnp.zeros((16,), jnp.float32)

    def _zero(k, carry):
        vbuf[pl.ds(k * 16, 16)] = zeros16
        return carry

    lax.fori_loop(0, _VBUF // 16, _zero, None)

    ones16 = jnp.ones((16,), jnp.float32)

    def _scatter_chunk(c, b, vals):
        for g in range(_ENT_PAD // 16):
            pos = pos_vmem[c, pl.ds(16 * g, 16)] + (b * _SLOT)
            plsc.store_scatter(vbuf, [pos], vals)

    def _chunk_pair(t, carry):
        for b in (0, 1):
            c = 2 * t + b
            elem0 = elem_base + c * _CHUNK_F32

            @pl.when(t >= 1)
            def _wait_and_clear(b=b, c=c, elem0=elem0):
                pltpu.make_async_copy(
                    vbuf.at[pl.ds(b * _SLOT, _CHUNK_F32)],
                    out_hbm.at[pl.ds(elem0 - 2 * _CHUNK_F32, _CHUNK_F32)],
                    sems[b],
                ).wait()
                _scatter_chunk(c - 2, b, zeros16)

            _scatter_chunk(c, b, ones16)
            pltpu.make_async_copy(
                vbuf.at[pl.ds(b * _SLOT, _CHUNK_F32)],
                out_hbm.at[pl.ds(elem0, _CHUNK_F32)],
                sems[b],
            ).start()
        return carry

    lax.fori_loop(0, _CHUNKS // 2, _chunk_pair, None)

    for b in (0, 1):
        pltpu.make_async_copy(
            vbuf.at[pl.ds(b * _SLOT, _CHUNK_F32)],
            out_hbm.at[pl.ds(elem_base, _CHUNK_F32)],
            sems[b],
        ).wait()


def kernel(inputs):
    b, w = inputs.shape
    idx32 = inputs.astype(jnp.int32)
    ent = jnp.arange(_ENT, dtype=jnp.int32)
    cell_base = (ent // _W) * _ROW + (ent % _W) * _VOCAB
    pos = idx32.reshape(_NW, _CHUNKS, _ENT) + cell_base[None, None, :]
    trash = _CHUNK_F32 + jnp.arange(_ENT_PAD - _ENT, dtype=jnp.int32)
    pos = jnp.concatenate(
        [pos, jnp.broadcast_to(trash, (_NW, _CHUNKS, _ENT_PAD - _ENT))], axis=-1
    )

    mesh = plsc.VectorSubcoreMesh(core_axis_name="c", subcore_axis_name="s")
    fn = pl.kernel(
        _sc_body,
        out_type=jax.ShapeDtypeStruct((b * _ROW,), jnp.float32),
        mesh=mesh,
        compiler_params=pltpu.CompilerParams(needs_layout_passes=False),
        scratch_types=[
            pltpu.VMEM((_CHUNKS, _ENT_PAD), jnp.int32),
            pltpu.VMEM((_VBUF,), jnp.float32),
            pltpu.SemaphoreType.DMA,
            pltpu.SemaphoreType.DMA,
        ],
    )
    return fn(pos).reshape(b, w, _VOCAB)


# speedup vs baseline: 1.0022x; 1.0022x over previous
import jax
import jax.numpy as jnp
from jax import lax
from jax.experimental import pallas as pl
from jax.experimental.pallas import tpu as pltpu
from jax.experimental.pallas import tpu_sc as plsc

_VOCAB = 1000
_W = 26
_ROW = _W * _VOCAB
_NC = 2
_NS = 16
_NW = _NC * _NS
_RPC = 2
_BATCH = 4096
_ROWS_PER_TILE = _BATCH // _NW
_CHUNKS = _ROWS_PER_TILE // _RPC
_ENT = _RPC * _W
_ENT_PAD = 64
_CHUNK_F32 = _RPC * _ROW
_SLOT = _CHUNK_F32 + 16
_VBUF = 2 * _SLOT


def _sc_body(pos_hbm, out_hbm, pos_vmem, vbuf, sem0, sem1):
    cid = lax.axis_index("c")
    sid = lax.axis_index("s")
    wid = sid * _NC + cid
    row_base = wid * _ROWS_PER_TILE
    sems = (sem0, sem1)

    pltpu.sync_copy(pos_hbm.at[wid], pos_vmem)

    zeros16 = jnp.zeros((16,), jnp.float32)

    def _zero_row(rr, carry):
        r0 = rr // _W
        r1 = rr - _W * r0

        def _zero_col(k, c2):
            vbuf[r0, r1, pl.ds(k * 16, 16)] = zeros16
            return c2

        lax.fori_loop(0, _VOCAB // 16, _zero_col, None)
        tail = jnp.full((16,), _VOCAB - 16, jnp.int32) + lax.iota(jnp.int32, 16)
        plsc.store_scatter(
            vbuf,
            [jnp.full((16,), r0, jnp.int32), jnp.full((16,), r1, jnp.int32), tail],
            zeros16,
        )
        return carry

    lax.fori_loop(0, 2 * _RPC * _W, _zero_row, None)

    ones16 = jnp.ones((16,), jnp.float32)

    def _scatter_chunk(c, b, vals):
        for g in range(_ENT_PAD // 16):
            e = lax.iota(jnp.int32, 16) + (16 * g)
            r = e // _W
            i0 = r + _RPC * b
            i1 = e - _W * r
            i2 = pos_vmem[c, pl.ds(16 * g, 16)]
            plsc.store_scatter(vbuf, [i0, i1, i2], vals, mask=e < _ENT)

    def _chunk_pair(t, carry):
        for b in (0, 1):
            c = 2 * t + b
            row0 = row_base + c * _RPC

            @pl.when(t >= 1)
            def _wait_and_clear(b=b, c=c, row0=row0):
                pltpu.make_async_copy(
                    vbuf.at[pl.ds(_RPC * b, _RPC)],
                    out_hbm.at[pl.ds(row0 - 2 * _RPC, _RPC)],
                    sems[b],
                ).wait()
                _scatter_chunk(c - 2, b, zeros16)

            _scatter_chunk(c, b, ones16)
            pltpu.make_async_copy(
                vbuf.at[pl.ds(_RPC * b, _RPC)],
                out_hbm.at[pl.ds(row0, _RPC)],
                sems[b],
            ).start()
        return carry

    lax.fori_loop(0, _CHUNKS // 2, _chunk_pair, None)

    for b in (0, 1):
        pltpu.make_async_copy(
            vbuf.at[pl.ds(_RPC * b, _RPC)],
            out_hbm.at[pl.ds(row_base, _RPC)],
            sems[b],
        ).wait()


def kernel(inputs):
    b, w = inputs.shape
    idx32 = inputs.astype(jnp.int32)
    pos = idx32.reshape(_NW, _CHUNKS, _ENT)
    pos = jnp.pad(pos, ((0, 0), (0, 0), (0, _ENT_PAD - _ENT)))

    mesh = plsc.VectorSubcoreMesh(core_axis_name="c", subcore_axis_name="s")
    fn = pl.kernel(
        _sc_body,
        out_type=jax.ShapeDtypeStruct((b, w, _VOCAB), jnp.float32),
        mesh=mesh,
        compiler_params=pltpu.CompilerParams(needs_layout_passes=False, use_tc_tiling_on_sc=False),
        scratch_types=[
            pltpu.VMEM((_CHUNKS, _ENT_PAD), jnp.int32),
            pltpu.VMEM((2 * _RPC, _W, _VOCAB), jnp.float32),
            pltpu.SemaphoreType.DMA,
            pltpu.SemaphoreType.DMA,
        ],
    )
    return fn(pos)


# speedup vs baseline: 1.9787x; 1.9743x over previous
import jax
import jax.numpy as jnp
from jax import lax
from jax.experimental import pallas as pl
from jax.experimental.pallas import tpu as pltpu
from jax.experimental.pallas import tpu_sc as plsc

_VOCAB = 1000
_W = 26
_ROW = _W * _VOCAB
_NC = 2
_NS = 16
_NW = _NC * _NS
_RPC = 1
_BATCH = 4096
_ROWS_PER_TILE = _BATCH // _NW
_CHUNKS = _ROWS_PER_TILE // _RPC
_ENT = _RPC * _W
_ENT_PAD = 32
_CHUNK_F32 = _RPC * _ROW
_SLOT = _CHUNK_F32 + 16
_VBUF = 2 * _SLOT


def _sc_body(pos_hbm, out_hbm, pos_vmem, vbuf, sem0, sem1):
    cid = lax.axis_index("c")
    sid = lax.axis_index("s")
    wid = sid * _NC + cid
    row_base = wid * _ROWS_PER_TILE
    sems = (sem0, sem1)

    pltpu.sync_copy(pos_hbm.at[wid], pos_vmem)

    zeros16 = jnp.zeros((16,), jnp.float32)

    def _zero_row(rr, carry):
        r0 = rr // _W
        r1 = rr - _W * r0

        def _zero_col(k, c2):
            vbuf[r0, r1, pl.ds(k * 16, 16)] = zeros16
            return c2

        lax.fori_loop(0, _VOCAB // 16, _zero_col, None)
        tail = jnp.full((16,), _VOCAB - 16, jnp.int32) + lax.iota(jnp.int32, 16)
        plsc.store_scatter(
            vbuf,
            [jnp.full((16,), r0, jnp.int32), jnp.full((16,), r1, jnp.int32), tail],
            zeros16,
        )
        return carry

    lax.fori_loop(0, 2 * _RPC * _W, _zero_row, None)

    ones16 = jnp.ones((16,), jnp.float32)

    def _scatter_chunk(c, b, vals):
        for g in range(_ENT_PAD // 16):
            e = lax.iota(jnp.int32, 16) + (16 * g)
            r = e // _W
            i0 = r + _RPC * b
            i1 = e - _W * r
            i2 = pos_vmem[c, pl.ds(16 * g, 16)]
            plsc.store_scatter(vbuf, [i0, i1, i2], vals, mask=e < _ENT)

    def _chunk_pair(t, carry):
        for b in (0, 1):
            c = 2 * t + b
            row0 = row_base + c * _RPC

            @pl.when(t >= 1)
            def _wait_and_clear(b=b, c=c, row0=row0):
                pltpu.make_async_copy(
                    vbuf.at[pl.ds(_RPC * b, _RPC)],
                    out_hbm.at[pl.ds(row0 - 2 * _RPC, _RPC)],
                    sems[b],
                ).wait()
                _scatter_chunk(c - 2, b, zeros16)

            _scatter_chunk(c, b, ones16)
            pltpu.make_async_copy(
                vbuf.at[pl.ds(_RPC * b, _RPC)],
                out_hbm.at[pl.ds(row0, _RPC)],
                sems[b],
            ).start()
        return carry

    lax.fori_loop(0, _CHUNKS // 2, _chunk_pair, None)

    for b in (0, 1):
        pltpu.make_async_copy(
            vbuf.at[pl.ds(_RPC * b, _RPC)],
            out_hbm.at[pl.ds(row_base, _RPC)],
            sems[b],
        ).wait()


def kernel(inputs):
    b, w = inputs.shape
    idx32 = inputs.astype(jnp.int32)
    pos = idx32.reshape(_NW, _CHUNKS, _ENT)
    pos = jnp.pad(pos, ((0, 0), (0, 0), (0, _ENT_PAD - _ENT)))

    mesh = plsc.VectorSubcoreMesh(core_axis_name="c", subcore_axis_name="s")
    fn = pl.kernel(
        _sc_body,
        out_type=jax.ShapeDtypeStruct((b, w, _VOCAB), jnp.float32),
        mesh=mesh,
        compiler_params=pltpu.CompilerParams(needs_layout_passes=False),
        scratch_types=[
            pltpu.VMEM((_CHUNKS, _ENT_PAD), jnp.int32),
            pltpu.VMEM((2 * _RPC, _W, _VOCAB), jnp.float32),
            pltpu.SemaphoreType.DMA,
            pltpu.SemaphoreType.DMA,
        ],
    )
    return fn(pos)
